# Initial kernel scaffold; baseline (speedup 1.0000x reference)
import jax
import jax.numpy as jnp
from jax.experimental import pallas as pl


def kernel(edge_vals, embed_user, embed_item, edge_user, edge_item):
    raise NotImplementedError("write your pallas kernel here")


# speedup vs baseline: 4.3371x; 4.3371x over previous
import functools

import jax
import jax.numpy as jnp
from jax import lax
from jax.experimental import pallas as pl
from jax.experimental.pallas import tpu as pltpu
from jax.experimental.pallas import tpu_sc as plsc

N_NODES = 50000
D_FULL = 64
DH = 32
E_RAW = 800000
NC = 2
NS = 16
BLK = 128
KB = 4
B_EDGES = KB * BLK
G_BATCHES = 100
E_PAD = NS * B_EDGES * G_BATCHES
N_ACC = 51200
ROWS_PER_TILE = N_ACC // NS
ZROWS = 400


def _spmm_body(src_hbm, sidx_hbm, didx_hbm, vals_hbm, out_hbm,
               sidx_v, didx_v, vals_v, rows_v, acc_sh, sem):
    c = lax.axis_index("c")
    s = lax.axis_index("s")

    @functools.partial(lax.fori_loop, 0, ZROWS, init_val=None)
    def _(i, _):
        zero16 = jnp.zeros((16,), jnp.float32)
        rows_v[i, pl.ds(0, 16)] = zero16
        rows_v[i, pl.ds(16, 16)] = zero16
        return None

    for r in range(ROWS_PER_TILE // ZROWS):
        pltpu.sync_copy(
            rows_v.at[pl.ds(0, ZROWS), :],
            acc_sh.at[pl.ds(s * ROWS_PER_TILE + r * ZROWS, ZROWS), :])
    plsc.subcore_barrier()

    off_vec = jnp.full((16,), c * N_ACC, dtype=jnp.int32)

    def batch_body(g, _):
        base_blk = (g * NS + s) * KB
        pltpu.sync_copy(sidx_hbm.at[pl.ds(base_blk, KB), :], sidx_v)
        pltpu.sync_copy(didx_hbm.at[pl.ds(base_blk, KB), :], didx_v)
        pltpu.sync_copy(vals_hbm.at[pl.ds(base_blk, KB), :], vals_v)

        for j in range(KB):
            for i8 in range(BLK // 16):
                sl = pl.ds(i8 * 16, 16)
                sidx_v[j, sl] = sidx_v[j, sl] + off_vec

        copies = [
            pltpu.async_copy(src_hbm.at[sidx_v.at[j]],
                             rows_v.at[pl.ds(j * BLK, BLK), :], sem)
            for j in range(KB)
        ]
        for cp in copies:
            cp.wait()

        for j in range(KB):
            @functools.partial(lax.fori_loop, 0, BLK // 16, init_val=None)
            def _(i16, _):
                vvec = vals_v[j, pl.ds(i16 * 16, 16)]
                for u in range(16):
                    e = j * BLK + i16 * 16 + u
                    v = vvec[u]
                    rows_v[e, pl.ds(0, 16)] = rows_v[e, pl.ds(0, 16)] * v
                    rows_v[e, pl.ds(16, 16)] = rows_v[e, pl.ds(16, 16)] * v
                return None

        for j in range(KB):
            pltpu.sync_copy(rows_v.at[pl.ds(j * BLK, BLK), :],
                            acc_sh.at[didx_v.at[j]], add=True)
        return None

    lax.fori_loop(0, G_BATCHES, batch_body, None)
    plsc.subcore_barrier()

    pltpu.sync_copy(acc_sh.at[pl.ds(s * ROWS_PER_TILE, ROWS_PER_TILE), :],
                    out_hbm.at[c, pl.ds(s * ROWS_PER_TILE, ROWS_PER_TILE), :])


_spmm = pl.kernel(
    _spmm_body,
    out_type=jax.ShapeDtypeStruct((NC, N_ACC, DH), jnp.float32),
    mesh=plsc.VectorSubcoreMesh(core_axis_name="c", subcore_axis_name="s"),
    scratch_types=[
        pltpu.VMEM((KB, BLK), jnp.int32),
        pltpu.VMEM((KB, BLK), jnp.int32),
        pltpu.VMEM((KB, BLK), jnp.float32),
        pltpu.VMEM((B_EDGES, DH), jnp.float32),
        pltpu.VMEM_SHARED((N_ACC, DH), jnp.float32),
        pltpu.SemaphoreType.DMA,
    ],
    compiler_params=pltpu.CompilerParams(use_tc_tiling_on_sc=False),
    name="bpr_spmm_sc",
)


def _combine_body(emb_ref, g1_ref, g2_ref, g3_ref, out_ref):
    lo = (emb_ref[:, 0:DH] + 0.5 * g1_ref[0] + (1.0 / 3.0) * g2_ref[0]
          + 0.25 * g3_ref[0])
    hi = (emb_ref[:, DH:D_FULL] + 0.5 * g1_ref[1] + (1.0 / 3.0) * g2_ref[1]
          + 0.25 * g3_ref[1])
    out_ref[:, 0:DH] = lo
    out_ref[:, DH:D_FULL] = hi


_COMBINE_ROWS = 2000


def _combine(emb, g1, g2, g3):
    grid = (N_NODES // _COMBINE_ROWS,)
    emb_spec = pl.BlockSpec((_COMBINE_ROWS, D_FULL), lambda i: (i, 0))
    g_spec = pl.BlockSpec((NC, _COMBINE_ROWS, DH), lambda i: (0, i, 0))
    return pl.pallas_call(
        _combine_body,
        grid=grid,
        in_specs=[emb_spec, g_spec, g_spec, g_spec],
        out_specs=emb_spec,
        out_shape=jax.ShapeDtypeStruct((N_NODES, D_FULL), jnp.float32),
    )(emb, g1, g2, g3)


def kernel(edge_vals, embed_user, embed_item, edge_user, edge_item):
    pad = E_PAD - E_RAW
    ev = jnp.concatenate(
        [edge_vals, jnp.zeros((pad,), jnp.float32)]).reshape(E_PAD // BLK, BLK)
    eu = jnp.concatenate(
        [edge_user, jnp.zeros((pad,), jnp.int32)]).reshape(E_PAD // BLK, BLK)
    ei = jnp.concatenate(
        [edge_item, jnp.zeros((pad,), jnp.int32)]).reshape(E_PAD // BLK, BLK)

    rpad = jnp.zeros((N_ACC - N_NODES, DH), jnp.float32)
    user_flat = jnp.concatenate(
        [embed_user[:, :DH], rpad, embed_user[:, DH:], rpad], axis=0)
    item_flat = jnp.concatenate(
        [embed_item[:, :DH], rpad, embed_item[:, DH:], rpad], axis=0)

    g1u = _spmm(item_flat, ei, eu, ev)
    g1i = _spmm(user_flat, eu, ei, ev)
    g2u = _spmm(g1i.reshape(NC * N_ACC, DH), ei, eu, ev)
    g2i = _spmm(g1u.reshape(NC * N_ACC, DH), eu, ei, ev)
    g3u = _spmm(g2i.reshape(NC * N_ACC, DH), ei, eu, ev)
    g3i = _spmm(g2u.reshape(NC * N_ACC, DH), eu, ei, ev)

    users = _combine(embed_user, g1u, g2u, g3u)
    items = _combine(embed_item, g1i, g2i, g3i)
    return (users, items)


# speedup vs baseline: 6.3154x; 1.4561x over previous
import functools

import jax
import jax.numpy as jnp
from jax import lax
from jax.experimental import pallas as pl
from jax.experimental.pallas import tpu as pltpu
from jax.experimental.pallas import tpu_sc as plsc

N_NODES = 50000
D_FULL = 64
DH = 32
E_RAW = 800000
NC = 2
NS = 16
BLK = 128
KB = 3
B_EDGES = KB * BLK
G_BATCHES = 132
E_PAD = NS * B_EDGES * G_BATCHES
N_ACC = 51200
ROWS_PER_TILE = N_ACC // NS
ZROWS = 320


def _scale_rows(rows_v, pck_v):
    for j in range(KB):
        @functools.partial(lax.fori_loop, 0, BLK // 16, init_val=None)
        def _(i16, _):
            vvec = plsc.bitcast(pck_v[2 * KB + j, pl.ds(i16 * 16, 16)],
                                jnp.float32)
            for u in range(16):
                e = j * BLK + i16 * 16 + u
                v = vvec[u]
                rows_v[e, pl.ds(0, 16)] = rows_v[e, pl.ds(0, 16)] * v
                rows_v[e, pl.ds(16, 16)] = rows_v[e, pl.ds(16, 16)] * v
            return None


def _spmm_body(src_hbm, pck_hbm, out_hbm,
               pck_a, pck_b, rows_a, rows_b, acc_sh, sem_a, sem_b, sem_s):
    c = lax.axis_index("c")
    s = lax.axis_index("s")

    @functools.partial(lax.fori_loop, 0, ZROWS, init_val=None)
    def _(i, _):
        zero16 = jnp.zeros((16,), jnp.float32)
        rows_a[i, pl.ds(0, 16)] = zero16
        rows_a[i, pl.ds(16, 16)] = zero16
        return None

    for r in range(ROWS_PER_TILE // ZROWS):
        pltpu.sync_copy(
            rows_a.at[pl.ds(0, ZROWS), :],
            acc_sh.at[pl.ds(s * ROWS_PER_TILE + r * ZROWS, ZROWS), :])
    plsc.subcore_barrier()

    off_vec = jnp.full((16,), c * N_ACC, dtype=jnp.int32)

    def load_and_offset(g, pck_v):
        pltpu.sync_copy(pck_hbm.at[g * NS + s], pck_v)
        for j in range(KB):
            for i8 in range(BLK // 16):
                sl = pl.ds(i8 * 16, 16)
                pck_v[j, sl] = pck_v[j, sl] + off_vec

    def fire_gathers(pck_v, rows_v, sem):
        for j in range(KB):
            pltpu.async_copy(src_hbm.at[pck_v.at[j]],
                             rows_v.at[pl.ds(j * BLK, BLK), :], sem)

    def wait_gathers(pck_v, rows_v, sem):
        for j in range(KB):
            pltpu.make_async_copy(src_hbm.at[pck_v.at[j]],
                                  rows_v.at[pl.ds(j * BLK, BLK), :], sem).wait()

    def scatter(pck_v, rows_v):
        cps = [
            pltpu.async_copy(rows_v.at[pl.ds(j * BLK, BLK), :],
                             acc_sh.at[pck_v.at[KB + j]], sem_s, add=True)
            for j in range(KB)
        ]
        for cp in cps:
            cp.wait()

    load_and_offset(0, pck_a)
    fire_gathers(pck_a, rows_a, sem_a)

    def pipe_body(h, _):
        g0 = 2 * h
        load_and_offset(g0 + 1, pck_b)
        wait_gathers(pck_a, rows_a, sem_a)
        fire_gathers(pck_b, rows_b, sem_b)
        _scale_rows(rows_a, pck_a)
        scatter(pck_a, rows_a)
        g2 = jnp.minimum(g0 + 2, G_BATCHES - 1)
        load_and_offset(g2, pck_a)
        wait_gathers(pck_b, rows_b, sem_b)
        fire_gathers(pck_a, rows_a, sem_a)
        _scale_rows(rows_b, pck_b)
        scatter(pck_b, rows_b)
        return None

    lax.fori_loop(0, G_BATCHES // 2, pipe_body, None)
    wait_gathers(pck_a, rows_a, sem_a)

    plsc.subcore_barrier()

    pltpu.sync_copy(acc_sh.at[pl.ds(s * ROWS_PER_TILE, ROWS_PER_TILE), :],
                    out_hbm.at[c, pl.ds(s * ROWS_PER_TILE, ROWS_PER_TILE), :])


_spmm = pl.kernel(
    _spmm_body,
    out_type=jax.ShapeDtypeStruct((NC, N_ACC, DH), jnp.float32),
    mesh=plsc.VectorSubcoreMesh(core_axis_name="c", subcore_axis_name="s"),
    scratch_types=[
        pltpu.VMEM((3 * KB, BLK), jnp.int32),
        pltpu.VMEM((3 * KB, BLK), jnp.int32),
        pltpu.VMEM((B_EDGES, DH), jnp.float32),
        pltpu.VMEM((B_EDGES, DH), jnp.float32),
        pltpu.VMEM_SHARED((N_ACC, DH), jnp.float32),
        pltpu.SemaphoreType.DMA,
        pltpu.SemaphoreType.DMA,
        pltpu.SemaphoreType.DMA,
    ],
    compiler_params=pltpu.CompilerParams(use_tc_tiling_on_sc=False, needs_layout_passes=False),
    name="bpr_spmm_sc",
)


def _combine_body(emb_ref, g1_ref, g2_ref, g3_ref, out_ref):
    lo = (emb_ref[:, 0:DH] + 0.5 * g1_ref[0] + (1.0 / 3.0) * g2_ref[0]
          + 0.25 * g3_ref[0])
    hi = (emb_ref[:, DH:D_FULL] + 0.5 * g1_ref[1] + (1.0 / 3.0) * g2_ref[1]
          + 0.25 * g3_ref[1])
    out_ref[:, 0:DH] = lo
    out_ref[:, DH:D_FULL] = hi


_COMBINE_ROWS = 2000


def _combine(emb, g1, g2, g3):
    grid = (N_NODES // _COMBINE_ROWS,)
    emb_spec = pl.BlockSpec((_COMBINE_ROWS, D_FULL), lambda i: (i, 0))
    g_spec = pl.BlockSpec((NC, _COMBINE_ROWS, DH), lambda i: (0, i, 0))
    return pl.pallas_call(
        _combine_body,
        grid=grid,
        in_specs=[emb_spec, g_spec, g_spec, g_spec],
        out_specs=emb_spec,
        out_shape=jax.ShapeDtypeStruct((N_NODES, D_FULL), jnp.float32),
    )(emb, g1, g2, g3)


def _pack(sidx, didx, vals):
    pad = E_PAD - E_RAW
    si = jnp.concatenate([sidx, jnp.zeros((pad,), jnp.int32)]
                         ).reshape(G_BATCHES * NS, KB, BLK)
    di = jnp.concatenate([didx, jnp.zeros((pad,), jnp.int32)]
                         ).reshape(G_BATCHES * NS, KB, BLK)
    ev = jnp.concatenate([vals, jnp.zeros((pad,), jnp.float32)]
                         ).reshape(G_BATCHES * NS, KB, BLK)
    evi = jax.lax.bitcast_convert_type(ev, jnp.int32)
    return jnp.concatenate([si, di, evi], axis=1)


def kernel(edge_vals, embed_user, embed_item, edge_user, edge_item):
    pck_u = _pack(edge_item, edge_user, edge_vals)
    pck_i = _pack(edge_user, edge_item, edge_vals)

    rpad = jnp.zeros((N_ACC - N_NODES, DH), jnp.float32)
    user_flat = jnp.concatenate(
        [embed_user[:, :DH], rpad, embed_user[:, DH:], rpad], axis=0)
    item_flat = jnp.concatenate(
        [embed_item[:, :DH], rpad, embed_item[:, DH:], rpad], axis=0)

    g1u = _spmm(item_flat, pck_u)
    g1i = _spmm(user_flat, pck_i)
    g2u = _spmm(g1i.reshape(NC * N_ACC, DH), pck_u)
    g2i = _spmm(g1u.reshape(NC * N_ACC, DH), pck_i)
    g3u = _spmm(g2i.reshape(NC * N_ACC, DH), pck_u)
    g3i = _spmm(g2u.reshape(NC * N_ACC, DH), pck_i)

    users = _combine(embed_user, g1u, g2u, g3u)
    items = _combine(embed_item, g1i, g2i, g3i)
    return (users, items)


# speedup vs baseline: 8.8564x; 1.4023x over previous
import functools

import jax
import jax.numpy as jnp
from jax import lax
from jax.experimental import pallas as pl
from jax.experimental.pallas import tpu as pltpu
from jax.experimental.pallas import tpu_sc as plsc

N_NODES = 50000
D_FULL = 64
DH = 32
E_RAW = 800000
NC = 2
NS = 16
BLK = 128
KB = 2
B_EDGES = KB * BLK
G_BATCHES = 196
E_PAD = NS * B_EDGES * G_BATCHES
N_ACC = 51200
ROWS_PER_TILE = N_ACC // NS
ZROWS = 200


def _spmm_body(src_hbm, pck_hbm, out_hbm,
               p0, p1, p2, p3, r0, r1, acc_sh,
               sp0, sp1, sp2, sp3, sg0, sg1, ss0, ss1):
    c = lax.axis_index("c")
    s = lax.axis_index("s")
    pbufs = [p0, p1, p2, p3]
    psems = [sp0, sp1, sp2, sp3]
    rbufs = [r0, r1]
    gsems = [sg0, sg1]
    ssems = [ss0, ss1]

    def zero_rows(rv, n):
        @functools.partial(lax.fori_loop, 0, n, init_val=None)
        def _(i, _):
            zero16 = jnp.zeros((16,), jnp.float32)
            rv[i, pl.ds(0, 16)] = zero16
            rv[i, pl.ds(16, 16)] = zero16
            return None

    zero_rows(r0, ZROWS)
    for rr in range(ROWS_PER_TILE // ZROWS):
        pltpu.sync_copy(
            r0.at[pl.ds(0, ZROWS), :],
            acc_sh.at[pl.ds(s * ROWS_PER_TILE + rr * ZROWS, ZROWS), :])
    plsc.subcore_barrier()

    def fire_load(g, pv, sem):
        b = jnp.minimum(g, G_BATCHES - 1) * NS + s
        pltpu.async_copy(pck_hbm.at[c, b], pv, sem)

    def wait_load(g, pv, sem):
        b = jnp.minimum(g, G_BATCHES - 1) * NS + s
        pltpu.make_async_copy(pck_hbm.at[c, b], pv, sem).wait()

    def fire_gathers(pv, rv, sem):
        for j in range(KB):
            pltpu.async_copy(src_hbm.at[pv.at[j]],
                             rv.at[pl.ds(j * BLK, BLK), :], sem)

    def wait_gathers(pv, rv, sem):
        for j in range(KB):
            pltpu.make_async_copy(src_hbm.at[pv.at[j]],
                                  rv.at[pl.ds(j * BLK, BLK), :], sem).wait()

    def fire_scatters(pv, rv, sem):
        for j in range(KB):
            pltpu.async_copy(rv.at[pl.ds(j * BLK, BLK), :],
                             acc_sh.at[pv.at[KB + j]], sem, add=True)

    def wait_scatters(pv, rv, sem):
        for j in range(KB):
            pltpu.make_async_copy(rv.at[pl.ds(j * BLK, BLK), :],
                                  acc_sh.at[pv.at[KB + j]], sem).wait()

    def scale_rows(rv, pv):
        for j in range(KB):
            @functools.partial(lax.fori_loop, 0, BLK // 16, init_val=None)
            def _(i16, _):
                vvec = plsc.bitcast(pv[2 * KB + j, pl.ds(i16 * 16, 16)],
                                    jnp.float32)
                for u in range(16):
                    e = j * BLK + i16 * 16 + u
                    v = vvec[u]
                    rv[e, pl.ds(0, 16)] = rv[e, pl.ds(0, 16)] * v
                    rv[e, pl.ds(16, 16)] = rv[e, pl.ds(16, 16)] * v
                return None

    fire_load(0, p0, sp0)
    fire_load(1, p1, sp1)
    zero_rows(r1, B_EDGES)
    wait_load(0, p0, sp0)
    fire_gathers(p0, r0, sg0)
    fire_scatters(p0, r1, ss1)

    def pipe_body(h, _):
        for q in range(4):
            g = 4 * h + q
            pv, pv1 = pbufs[q], pbufs[(q + 1) % 4]
            pv2 = pbufs[(q + 2) % 4]
            rv, rv1 = rbufs[q % 2], rbufs[(q + 1) % 2]
            fire_load(g + 2, pv2, psems[(q + 2) % 4])
            wait_gathers(pv, rv, gsems[q % 2])
            wait_scatters(pv1, rv1, ssems[(q + 1) % 2])
            wait_load(g + 1, pv1, psems[(q + 1) % 4])
            fire_gathers(pv1, rv1, gsems[(q + 1) % 2])
            scale_rows(rv, pv)
            fire_scatters(pv, rv, ssems[q % 2])
        return None

    lax.fori_loop(0, G_BATCHES // 4, pipe_body, None)

    wait_load(G_BATCHES + 1, p1, sp1)
    wait_gathers(p0, r0, sg0)
    wait_scatters(p1, r1, ss1)

    plsc.subcore_barrier()

    pltpu.sync_copy(acc_sh.at[pl.ds(s * ROWS_PER_TILE, ROWS_PER_TILE), :],
                    out_hbm.at[c, pl.ds(s * ROWS_PER_TILE, ROWS_PER_TILE), :])


_spmm = pl.kernel(
    _spmm_body,
    out_type=jax.ShapeDtypeStruct((NC, N_ACC, DH), jnp.float32),
    mesh=plsc.VectorSubcoreMesh(core_axis_name="c", subcore_axis_name="s"),
    scratch_types=(
        [pltpu.VMEM((3 * KB, BLK), jnp.int32) for _ in range(4)]
        + [pltpu.VMEM((B_EDGES, DH), jnp.float32) for _ in range(2)]
        + [pltpu.VMEM_SHARED((N_ACC, DH), jnp.float32)]
        + [pltpu.SemaphoreType.DMA] * 8
    ),
    compiler_params=pltpu.CompilerParams(use_tc_tiling_on_sc=False,
                                         needs_layout_passes=False),
    name="bpr_spmm_sc",
)


def _combine_body(emb_ref, g1_ref, g2_ref, g3_ref, out_ref):
    lo = (emb_ref[:, 0:DH] + 0.5 * g1_ref[0] + (1.0 / 3.0) * g2_ref[0]
          + 0.25 * g3_ref[0])
    hi = (emb_ref[:, DH:D_FULL] + 0.5 * g1_ref[1] + (1.0 / 3.0) * g2_ref[1]
          + 0.25 * g3_ref[1])
    out_ref[:, 0:DH] = lo
    out_ref[:, DH:D_FULL] = hi


_COMBINE_ROWS = 2000


def _combine(emb, g1, g2, g3):
    grid = (N_NODES // _COMBINE_ROWS,)
    emb_spec = pl.BlockSpec((_COMBINE_ROWS, D_FULL), lambda i: (i, 0))
    g_spec = pl.BlockSpec((NC, _COMBINE_ROWS, DH), lambda i: (0, i, 0))
    return pl.pallas_call(
        _combine_body,
        grid=grid,
        in_specs=[emb_spec, g_spec, g_spec, g_spec],
        out_specs=emb_spec,
        out_shape=jax.ShapeDtypeStruct((N_NODES, D_FULL), jnp.float32),
    )(emb, g1, g2, g3)


def _pack(sidx, didx, vals):
    pad = E_PAD - E_RAW
    si = jnp.concatenate([sidx, jnp.zeros((pad,), jnp.int32)]
                         ).reshape(G_BATCHES * NS, KB, BLK)
    di = jnp.concatenate([didx, jnp.zeros((pad,), jnp.int32)]
                         ).reshape(G_BATCHES * NS, KB, BLK)
    ev = jnp.concatenate([vals, jnp.zeros((pad,), jnp.float32)]
                         ).reshape(G_BATCHES * NS, KB, BLK)
    evi = jax.lax.bitcast_convert_type(ev, jnp.int32)
    return jnp.stack(
        [jnp.concatenate([si + cc * N_ACC, di, evi], axis=1)
         for cc in range(NC)])


def kernel(edge_vals, embed_user, embed_item, edge_user, edge_item):
    pck_u = _pack(edge_item, edge_user, edge_vals)
    pck_i = _pack(edge_user, edge_item, edge_vals)

    rpad = jnp.zeros((N_ACC - N_NODES, DH), jnp.float32)
    user_flat = jnp.concatenate(
        [embed_user[:, :DH], rpad, embed_user[:, DH:], rpad], axis=0)
    item_flat = jnp.concatenate(
        [embed_item[:, :DH], rpad, embed_item[:, DH:], rpad], axis=0)

    g1u = _spmm(item_flat, pck_u)
    g1i = _spmm(user_flat, pck_i)
    g2u = _spmm(g1i.reshape(NC * N_ACC, DH), pck_u)
    g2i = _spmm(g1u.reshape(NC * N_ACC, DH), pck_i)
    g3u = _spmm(g2i.reshape(NC * N_ACC, DH), pck_u)
    g3i = _spmm(g2u.reshape(NC * N_ACC, DH), pck_i)

    users = _combine(embed_user, g1u, g2u, g3u)
    items = _combine(embed_item, g1i, g2i, g3i)
    return (users, items)
